# Initial kernel scaffold; baseline (speedup 1.0000x reference)
import jax
import jax.numpy as jnp
from jax.experimental import pallas as pl


def kernel(x, adj, W_att, a_att, W_out, a_out):
    raise NotImplementedError("write your pallas kernel here")


# speedup vs baseline: 4.8689x; 4.8689x over previous
import functools

import jax
import jax.numpy as jnp
from jax import lax
from jax.experimental import pallas as pl
from jax.experimental.pallas import tpu as pltpu
from jax.experimental.pallas import tpu_sc as plsc

N = 10000
E = 160000
F_IN = 256
NHID = 64
NH = 8
NLABEL = 121
ALPHA = 0.2

NC = 2
NS = 16
LANES = 16
B = 80
AW = 144
NPS = 624
EP = 160256

f32 = jnp.float32


_R = 1000


def _elu(v):
    return jnp.where(v > 0, v, jnp.exp(jnp.minimum(v, 0.0)) - 1.0)


def _tc1_body(x_ref, wall_ref, smat_ref, h0, h1, h2, h3, s_ref):
    h = jnp.dot(x_ref[:], wall_ref[:], preferred_element_type=f32)
    s_ref[:] = jnp.dot(h, smat_ref[:], preferred_element_type=f32)
    h0[:] = h[:, 0:128]
    h1[:] = h[:, 128:256]
    h2[:] = h[:, 256:384]
    h3[:] = h[:, 384:512]


def _tc1(x, wall, smat):
    grid = (N // _R,)
    return pl.pallas_call(
        _tc1_body,
        grid=grid,
        in_specs=[
            pl.BlockSpec((_R, F_IN), lambda i: (i, 0)),
            pl.BlockSpec((F_IN, NH * NHID), lambda i: (0, 0)),
            pl.BlockSpec((NH * NHID, 16), lambda i: (0, 0)),
        ],
        out_specs=[pl.BlockSpec((_R, 128), lambda i: (i, 0))] * 4
        + [pl.BlockSpec((_R, 16), lambda i: (i, 0))],
        out_shape=[jax.ShapeDtypeStruct((N, 128), f32)] * 4
        + [jax.ShapeDtypeStruct((N, 16), f32)],
    )(x, wall, smat)


def _tc2_body(o0, o1, o2, o3, wbig_ref, out_ref):
    acc = jnp.zeros((_R, AW), f32)
    col = lax.broadcasted_iota(jnp.int32, (_R, AW), 1)
    for c, o in enumerate((o0, o1, o2, o3)):
        a = o[:]
        rs = jnp.where(col < 64, a[:, 128:129], a[:, 129:130])
        hc = _elu(a / rs)
        acc = acc + jnp.dot(hc[:, 0:128], wbig_ref[pl.ds(c * 128, 128), :],
                            preferred_element_type=f32)
    out_ref[:] = acc


def _tc2(o0, o1, o2, o3, wbig):
    grid = (N // _R,)
    return pl.pallas_call(
        _tc2_body,
        grid=grid,
        in_specs=[pl.BlockSpec((_R, AW), lambda i: (i, 0))] * 4
        + [pl.BlockSpec((NH * NHID, AW), lambda i: (0, 0))],
        out_specs=pl.BlockSpec((_R, AW), lambda i: (i, 0)),
        out_shape=jax.ShapeDtypeStruct((N, AW), f32),
    )(o0, o1, o2, o3, wbig)


def _tc3_body(p0, p1, out_ref):
    a = p0[:] + p1[:]
    v = a / a[:, 128:129]
    out_ref[:] = jax.nn.sigmoid(_elu(v))


def _tc3(p0, p1):
    grid = (N // _R,)
    return pl.pallas_call(
        _tc3_body,
        grid=grid,
        in_specs=[pl.BlockSpec((_R, AW), lambda i: (i, 0))] * 2,
        out_specs=pl.BlockSpec((_R, AW), lambda i: (i, 0)),
        out_shape=jax.ShapeDtypeStruct((N, AW), f32),
    )(p0, p1)


_MESH = plsc.VectorSubcoreMesh(
    core_axis_name="c", subcore_axis_name="s", num_cores=NC, num_subcores=NS)


def _edge_weight(e):
    return jnp.exp(-jnp.maximum(e, ALPHA * e))


def _zero_wbuf_tail(wbuf):
    def zrow(r, carry):
        wbuf[r, pl.ds(128, 16)] = jnp.zeros((16,), f32)
        return carry
    lax.fori_loop(0, B, zrow, None)


def _splat(r):
    return jnp.full((16,), 0, jnp.int32) + r


def _node_copy(src, dst, sid):
    row0 = pl.multiple_of(sid * NPS, 8)
    pltpu.sync_copy(src.at[pl.ds(row0, NPS)], dst.at[pl.ds(row0, NPS)])

    @pl.when(sid == NS - 1)
    def _():
        pltpu.sync_copy(src.at[pl.ds(NS * NPS, N - NS * NPS)],
                        dst.at[pl.ds(NS * NPS, N - NS * NPS)])


def _scale_rows(gbuf, wbuf, wv0, wv1):
    def srow(r, carry):
        a0 = plsc.load_gather(wv0, [_splat(r)])
        a1 = plsc.load_gather(wv1, [_splat(r)])
        for j in range(4):
            wbuf[r, pl.ds(j * 16, 16)] = gbuf[r, pl.ds(j * 16, 16)] * a0
        for j in range(4, 8):
            wbuf[r, pl.ds(j * 16, 16)] = gbuf[r, pl.ds(j * 16, 16)] * a1
        return carry
    lax.fori_loop(0, B, srow, None, unroll=2)


@functools.partial(
    pl.kernel,
    out_type=[jax.ShapeDtypeStruct((N, AW), f32)] * 2,
    mesh=_MESH,
    compiler_params=pltpu.CompilerParams(use_tc_tiling_on_sc=False, needs_layout_passes=False),
    scratch_types=[
        pltpu.VMEM((B,), jnp.int32),
        pltpu.VMEM((B,), jnp.int32),
        pltpu.VMEM((B,), f32),
        pltpu.VMEM((B,), f32),
        pltpu.VMEM((B,), f32),
        pltpu.VMEM((B,), f32),
        pltpu.VMEM((B,), f32),
        pltpu.VMEM((B,), f32),
        pltpu.VMEM((B, 128), f32),
        pltpu.VMEM((B, AW), f32),
        pltpu.VMEM_SHARED((N, AW), f32),
        pltpu.SemaphoreType.DMA,
        pltpu.SemaphoreType.DMA,
    ],
)
def _sc_layer1(hA, hB,
               sA0, sA1, sA2, sA3, sB0, sB1, sB2, sB3,
               srcp, dstp, zer, oA, oB,
               srcb, dstb, s1av, s1bv, s2av, s2bv, wv0, wv1, gbuf, wbuf,
               acc, semI, semG):
    cid = lax.axis_index("c")
    sid = lax.axis_index("s")
    _zero_wbuf_tail(wbuf)
    htabs = (hA, hB)
    outs = (oA, oB)
    svecs = ((sA0, sA1, sA2, sA3), (sB0, sB1, sB2, sB3))
    for k in range(NC):
        @pl.when(cid == k)
        def _(k=k):
            htab = htabs[k]
            out = outs[k]
            sv4 = svecs[k]
            _node_copy(zer, acc, sid)
            plsc.subcore_barrier()
            ebase = sid * (E // NS)

            def batch(b, carry):
                base = pl.multiple_of(ebase + b * B, 8)
                d1 = pltpu.async_copy(srcp.at[pl.ds(base, B)], srcb, semI)
                d2 = pltpu.async_copy(dstp.at[pl.ds(base, B)], dstb, semI)
                d1.wait()
                d2.wait()
                g0 = pltpu.async_copy(htab.at[dstb], gbuf, semG)
                g1 = pltpu.async_copy(sv4[0].at[srcb], s1av, semG)
                g2 = pltpu.async_copy(sv4[1].at[srcb], s1bv, semG)
                g3 = pltpu.async_copy(sv4[2].at[dstb], s2av, semG)
                g4 = pltpu.async_copy(sv4[3].at[dstb], s2bv, semG)
                g0.wait()
                g1.wait()
                g2.wait()
                g3.wait()
                g4.wait()
                for g in range(B // 16):
                    sl = pl.ds(g * 16, 16)
                    w0 = _edge_weight(s1av[sl] + s2av[sl])
                    w1 = _edge_weight(s1bv[sl] + s2bv[sl])
                    wv0[sl] = w0
                    wv1[sl] = w1
                    rows = lax.iota(jnp.int32, 16) + g * 16
                    plsc.store_scatter(
                        wbuf, [rows, jnp.full((16,), 128, jnp.int32)], w0)
                    plsc.store_scatter(
                        wbuf, [rows, jnp.full((16,), 129, jnp.int32)], w1)
                _scale_rows(gbuf, wbuf, wv0, wv1)
                pltpu.sync_copy(wbuf, acc.at[srcb], add=True)
                return carry

            lax.fori_loop(0, E // NS // B, batch, None)
            plsc.subcore_barrier()
            _node_copy(acc, out, sid)
            plsc.subcore_barrier()


@functools.partial(
    pl.kernel,
    out_type=[jax.ShapeDtypeStruct((N, AW), f32)] * 2,
    mesh=_MESH,
    compiler_params=pltpu.CompilerParams(use_tc_tiling_on_sc=False, needs_layout_passes=False),
    scratch_types=[
        pltpu.VMEM((B,), jnp.int32),
        pltpu.VMEM((B,), jnp.int32),
        pltpu.VMEM((B,), f32),
        pltpu.VMEM((B,), f32),
        pltpu.VMEM((B, AW), f32),
        pltpu.VMEM((B, AW), f32),
        pltpu.VMEM_SHARED((N, AW), f32),
        pltpu.SemaphoreType.DMA,
        pltpu.SemaphoreType.DMA,
    ],
)
def _sc_layer2(h2b, s1r, srcp, dstp, zer, p0, p1,
               srcb, dstb, s1v, wv0, gbuf, wbuf, acc, semI, semG):
    cid = lax.axis_index("c")
    sid = lax.axis_index("s")
    _zero_wbuf_tail(wbuf)
    _node_copy(zer, acc, sid)
    plsc.subcore_barrier()
    epc = E // NC
    ept = epc // NS
    nb = (ept + B - 1) // B

    def batch(b, carry):
        base = pl.multiple_of(cid * epc + sid * ept + b * B, 8)
        d1 = pltpu.async_copy(srcp.at[pl.ds(base, B)], srcb, semI)
        d2 = pltpu.async_copy(dstp.at[pl.ds(base, B)], dstb, semI)
        d1.wait()
        d2.wait()
        g0 = pltpu.async_copy(h2b.at[dstb], gbuf, semG)
        g1 = pltpu.async_copy(s1r.at[srcb], s1v, semG)
        g0.wait()
        g1.wait()
        thresh = jnp.minimum(B, ept - b * B)
        for g in range(B // 16):
            sl = pl.ds(g * 16, 16)
            rows = lax.iota(jnp.int32, 16) + g * 16
            x1 = s1v[sl]
            x2 = plsc.load_gather(gbuf, [rows, jnp.full((16,), 128, jnp.int32)])
            w = _edge_weight(x1 + x2)
            w = jnp.where(rows < thresh, w, jnp.zeros((16,), f32))
            wv0[pl.ds(g * 16, 16)] = w
            plsc.store_scatter(
                wbuf, [rows, jnp.full((16,), 128, jnp.int32)], w)

        def srow(r, c2):
            a0 = plsc.load_gather(wv0, [_splat(r)])
            for j in range(8):
                wbuf[r, pl.ds(j * 16, 16)] = gbuf[r, pl.ds(j * 16, 16)] * a0
            return c2

        lax.fori_loop(0, B, srow, None, unroll=2)
        pltpu.sync_copy(wbuf, acc.at[srcb], add=True)
        return carry

    lax.fori_loop(0, nb, batch, None)
    plsc.subcore_barrier()
    for k in range(NC):
        @pl.when(cid == k)
        def _(k=k):
            _node_copy(acc, (p0, p1)[k], sid)


def kernel(x, adj, W_att, a_att, W_out, a_out):
    src = adj[0]
    dst = adj[1]
    pad = jnp.zeros((EP - E,), jnp.int32)
    srcp = jnp.concatenate([src, pad])
    dstp = jnp.concatenate([dst, pad])

    wall = jnp.transpose(W_att, (1, 0, 2)).reshape(F_IN, NH * NHID)
    a1 = a_att[:, 0, :NHID]
    a2 = a_att[:, 0, NHID:]
    eye = jnp.eye(NH, dtype=f32)
    s1m = (a1[:, :, None] * eye[:, None, :]).reshape(NH * NHID, NH)
    s2m = (a2[:, :, None] * eye[:, None, :]).reshape(NH * NHID, NH)
    smat = jnp.concatenate([s1m, s2m], axis=1)

    v1 = W_out @ a_out[0, :NLABEL]
    v2 = W_out @ a_out[0, NLABEL:]
    wbig = jnp.concatenate(
        [W_out, jnp.zeros((NH * NHID, 7), f32), v2[:, None], v1[:, None],
         jnp.zeros((NH * NHID, AW - 130), f32)], axis=1)

    zer = jnp.zeros((N, AW), f32)

    h0, h1, h2, h3, s = _tc1(x, wall, smat)
    sts = [s[:, i] for i in range(16)]
    o0, o1 = _sc_layer1(h0, h1, sts[0], sts[1], sts[8], sts[9],
                        sts[2], sts[3], sts[10], sts[11], srcp, dstp, zer)
    o2, o3 = _sc_layer1(h2, h3, sts[4], sts[5], sts[12], sts[13],
                        sts[6], sts[7], sts[14], sts[15], srcp, dstp, zer)
    h2b = _tc2(o0, o1, o2, o3, wbig)
    s1r = h2b[:, 129]
    p0, p1 = _sc_layer2(h2b, s1r, srcp, dstp, zer)
    full = _tc3(p0, p1)
    return full[:, :NLABEL]


# speedup vs baseline: 14.0952x; 2.8949x over previous
import functools

import jax
import jax.numpy as jnp
from jax import lax
from jax.experimental import pallas as pl
from jax.experimental.pallas import tpu as pltpu
from jax.experimental.pallas import tpu_sc as plsc

N = 10000
E = 160000
F_IN = 256
NHID = 64
NH = 8
NLABEL = 121
ALPHA = 0.2

NC = 2
NS = 16
B = 80
NPS = 624
EP = 160256

f32 = jnp.float32
i32 = jnp.int32

_SC_PARAMS = pltpu.CompilerParams(
    use_tc_tiling_on_sc=False, needs_layout_passes=False)


_R = 1000


def _elu(v):
    return jnp.where(v > 0, v, jnp.exp(jnp.minimum(v, 0.0)) - 1.0)


def _tc1_body(x_ref, wall_ref, smat_ref, h0, h1, h2, h3, s_ref):
    h = jnp.dot(x_ref[:], wall_ref[:], preferred_element_type=f32)
    s_ref[:] = jnp.dot(h, smat_ref[:], preferred_element_type=f32)
    h0[:] = h[:, 0:128]
    h1[:] = h[:, 128:256]
    h2[:] = h[:, 256:384]
    h3[:] = h[:, 384:512]


def _tc1(x, wall, smat):
    grid = (N // _R,)
    return pl.pallas_call(
        _tc1_body,
        grid=grid,
        in_specs=[
            pl.BlockSpec((_R, F_IN), lambda i: (i, 0)),
            pl.BlockSpec((F_IN, NH * NHID), lambda i: (0, 0)),
            pl.BlockSpec((NH * NHID, 16), lambda i: (0, 0)),
        ],
        out_specs=[pl.BlockSpec((_R, 128), lambda i: (i, 0))] * 4
        + [pl.BlockSpec((_R, 16), lambda i: (i, 0))],
        out_shape=[jax.ShapeDtypeStruct((N, 128), f32)] * 4
        + [jax.ShapeDtypeStruct((N, 16), f32)],
    )(x, wall, smat)


def _tc2_body(f0, f1, f2, f3, r0, r1, r2, r3, wbig_ref, h2p_ref, sv_ref):
    acc = jnp.zeros((_R, 144), f32)
    col = lax.broadcasted_iota(i32, (_R, 128), 1)
    for c, (fo, ro) in enumerate(zip((f0, f1, f2, f3), (r0, r1, r2, r3))):
        a = fo[:]
        r = ro[:]
        rs = jnp.where(col < 64, r[:, 0:1], r[:, 1:2])
        hc = _elu(a / rs)
        acc = acc + jnp.dot(hc, wbig_ref[pl.ds(c * 128, 128), :],
                            preferred_element_type=f32)
    h2p_ref[:] = acc[:, 0:128]
    sv_ref[:] = acc[:, 128:144]


def _tc2(f0, f1, f2, f3, r0, r1, r2, r3, wbig):
    grid = (N // _R,)
    return pl.pallas_call(
        _tc2_body,
        grid=grid,
        in_specs=[pl.BlockSpec((_R, 128), lambda i: (i, 0))] * 4
        + [pl.BlockSpec((_R, 16), lambda i: (i, 0))] * 4
        + [pl.BlockSpec((NH * NHID, 144), lambda i: (0, 0))],
        out_specs=[pl.BlockSpec((_R, 128), lambda i: (i, 0)),
                   pl.BlockSpec((_R, 16), lambda i: (i, 0))],
        out_shape=[jax.ShapeDtypeStruct((N, 128), f32),
                   jax.ShapeDtypeStruct((N, 16), f32)],
    )(f0, f1, f2, f3, r0, r1, r2, r3, wbig)


def _tc3_body(f0, f1, r0, r1, out_ref):
    a = f0[:] + f1[:]
    rs = r0[:, 0:1] + r1[:, 0:1]
    out_ref[:] = jax.nn.sigmoid(_elu(a / rs))


def _tc3(f0, f1, r0, r1):
    grid = (N // _R,)
    return pl.pallas_call(
        _tc3_body,
        grid=grid,
        in_specs=[pl.BlockSpec((_R, 128), lambda i: (i, 0))] * 2
        + [pl.BlockSpec((_R, 16), lambda i: (i, 0))] * 2,
        out_specs=pl.BlockSpec((_R, 128), lambda i: (i, 0)),
        out_shape=jax.ShapeDtypeStruct((N, 128), f32),
    )(f0, f1, r0, r1)


_MESH = plsc.VectorSubcoreMesh(
    core_axis_name="c", subcore_axis_name="s", num_cores=NC, num_subcores=NS)


def _edge_weight(e):
    return jnp.exp(-jnp.maximum(e, ALPHA * e))


def _splat(r):
    return jnp.full((16,), 0, i32) + r


def _zero_rows(buf, nrows):
    def zrow(r, carry):
        buf[r, pl.ds(0, 16)] = jnp.zeros((16,), f32)
        return carry
    lax.fori_loop(0, nrows, zrow, None)


def _node_copy(src, dst, sid):
    row0 = pl.multiple_of(sid * NPS, 8)
    pltpu.sync_copy(src.at[pl.ds(row0, NPS)], dst.at[pl.ds(row0, NPS)])

    @pl.when(sid == NS - 1)
    def _():
        pltpu.sync_copy(src.at[pl.ds(NS * NPS, N - NS * NPS)],
                        dst.at[pl.ds(NS * NPS, N - NS * NPS)])


def _copy_idx(dst, src):
    for t in range(B // 16):
        sl = pl.ds(t * 16, 16)
        dst[sl] = src[sl]


@functools.partial(
    pl.kernel,
    out_type=[jax.ShapeDtypeStruct((N, 128), f32),
              jax.ShapeDtypeStruct((N, 16), f32)] * 2,
    mesh=_MESH,
    compiler_params=_SC_PARAMS,
    scratch_types=[
        pltpu.VMEM((B,), i32),
        pltpu.VMEM((B,), i32),
        pltpu.VMEM((B,), f32),
        pltpu.VMEM((B,), f32),
        pltpu.VMEM((B,), f32),
        pltpu.VMEM((B,), f32),
        pltpu.VMEM((B, 128), f32),
        pltpu.VMEM((B,), i32),
        pltpu.VMEM((B,), i32),
        pltpu.VMEM((B,), f32),
        pltpu.VMEM((B,), f32),
        pltpu.VMEM((B,), f32),
        pltpu.VMEM((B,), f32),
        pltpu.VMEM((B, 128), f32),
        pltpu.VMEM((B,), f32),
        pltpu.VMEM((B,), f32),
        pltpu.VMEM((B,), i32),
        pltpu.VMEM((B, 16), f32),
        pltpu.VMEM_SHARED((N, 128), f32),
        pltpu.VMEM_SHARED((N, 16), f32),
        pltpu.SemaphoreType.DMA,
        pltpu.SemaphoreType.DMA,
        pltpu.SemaphoreType.DMA,
        pltpu.SemaphoreType.DMA,
    ],
)
def _sc_layer1(hA, hB,
               sA0, sA1, sA2, sA3, sB0, sB1, sB2, sB3,
               srcp, dstp, zf, zr, oAf, oAr, oBf, oBr,
               srcb0, dstb0, s1a0, s1b0, s2a0, s2b0, gbuf0,
               srcb1, dstb1, s1a1, s1b1, s2a1, s2b1, gbuf1,
               wv0, wv1, sbuf, wrbuf,
               acc, accr, semI0, semI1, semG0, semG1):
    cid = lax.axis_index("c")
    sid = lax.axis_index("s")
    _zero_rows(wrbuf, B)
    htabs = (hA, hB)
    fouts = (oAf, oBf)
    routs = (oAr, oBr)
    svecs = ((sA0, sA1, sA2, sA3), (sB0, sB1, sB2, sB3))
    S0 = (srcb0, dstb0, s1a0, s1b0, s2a0, s2b0, gbuf0, semI0, semG0)
    S1 = (srcb1, dstb1, s1a1, s1b1, s2a1, s2b1, gbuf1, semI1, semG1)
    NB = E // NS // B

    for k in range(NC):
        @pl.when(cid == k)
        def _(k=k):
            htab = htabs[k]
            sv4 = svecs[k]
            _node_copy(zf, acc, sid)
            _node_copy(zr, accr, sid)
            plsc.subcore_barrier()
            ebase = sid * (E // NS)

            def base_of(b):
                return pl.multiple_of(ebase + b * B, 8)

            def idx_issue(b, s):
                base = base_of(b)
                pltpu.async_copy(srcp.at[pl.ds(base, B)], s[0], s[7])
                pltpu.async_copy(dstp.at[pl.ds(base, B)], s[1], s[7])

            def idx_wait(b, s):
                base = base_of(b)
                pltpu.make_async_copy(srcp.at[pl.ds(base, B)], s[0], s[7]).wait()
                pltpu.make_async_copy(dstp.at[pl.ds(base, B)], s[1], s[7]).wait()

            def g_issue(s):
                srcb, dstb, s1a, s1b, s2a, s2b, gbuf, _, semG = s
                pltpu.async_copy(htab.at[dstb], gbuf, semG)
                pltpu.async_copy(sv4[0].at[srcb], s1a, semG)
                pltpu.async_copy(sv4[1].at[srcb], s1b, semG)
                pltpu.async_copy(sv4[2].at[dstb], s2a, semG)
                pltpu.async_copy(sv4[3].at[dstb], s2b, semG)

            def g_wait(s):
                srcb, dstb, s1a, s1b, s2a, s2b, gbuf, _, semG = s
                pltpu.make_async_copy(htab.at[dstb], gbuf, semG).wait()
                pltpu.make_async_copy(sv4[0].at[srcb], s1a, semG).wait()
                pltpu.make_async_copy(sv4[1].at[srcb], s1b, semG).wait()
                pltpu.make_async_copy(sv4[2].at[dstb], s2a, semG).wait()
                pltpu.make_async_copy(sv4[3].at[dstb], s2b, semG).wait()

            def process(s):
                _, _, s1a, s1b, s2a, s2b, gbuf, _, _ = s
                for g in range(B // 16):
                    sl = pl.ds(g * 16, 16)
                    rows = lax.iota(i32, 16) + g * 16
                    w0 = _edge_weight(s1a[sl] + s2a[sl])
                    w1 = _edge_weight(s1b[sl] + s2b[sl])
                    wv0[sl] = w0
                    wv1[sl] = w1
                    plsc.store_scatter(wrbuf, [rows, _splat(0)], w0)
                    plsc.store_scatter(wrbuf, [rows, _splat(1)], w1)

                def srow(r, carry):
                    a0 = plsc.load_gather(wv0, [_splat(r)])
                    a1 = plsc.load_gather(wv1, [_splat(r)])
                    for j in range(4):
                        sl = pl.ds(j * 16, 16)
                        gbuf[r, sl] = gbuf[r, sl] * a0
                    for j in range(4, 8):
                        sl = pl.ds(j * 16, 16)
                        gbuf[r, sl] = gbuf[r, sl] * a1
                    return carry

                lax.fori_loop(0, B, srow, None, unroll=4)
                pltpu.sync_copy(gbuf, acc.at[sbuf], add=True)
                pltpu.sync_copy(wrbuf, accr.at[sbuf], add=True)

            idx_issue(0, S0)
            idx_issue(1, S1)
            idx_wait(0, S0)
            g_issue(S0)

            def pair(i, carry):
                b0 = 2 * i
                g_wait(S0)
                _copy_idx(sbuf, S0[0])
                idx_issue(b0 + 2, S0)
                idx_wait(b0 + 1, S1)
                g_issue(S1)
                process(S0)
                g_wait(S1)
                _copy_idx(sbuf, S1[0])
                idx_issue(b0 + 3, S1)
                idx_wait(b0 + 2, S0)
                g_issue(S0)
                process(S1)
                return carry

            lax.fori_loop(0, NB // 2, pair, None)
            g_wait(S0)
            _copy_idx(sbuf, S0[0])
            process(S0)
            idx_wait(NB, S1)
            plsc.subcore_barrier()
            _node_copy(acc, fouts[k], sid)
            _node_copy(accr, routs[k], sid)
            plsc.subcore_barrier()


@functools.partial(
    pl.kernel,
    out_type=[jax.ShapeDtypeStruct((N, 128), f32),
              jax.ShapeDtypeStruct((N, 16), f32)] * 2,
    mesh=_MESH,
    compiler_params=_SC_PARAMS,
    scratch_types=[
        pltpu.VMEM((B,), i32),
        pltpu.VMEM((B,), i32),
        pltpu.VMEM((B,), f32),
        pltpu.VMEM((B,), f32),
        pltpu.VMEM((B, 128), f32),
        pltpu.VMEM((B,), i32),
        pltpu.VMEM((B,), i32),
        pltpu.VMEM((B,), f32),
        pltpu.VMEM((B,), f32),
        pltpu.VMEM((B, 128), f32),
        pltpu.VMEM((B,), f32),
        pltpu.VMEM((B,), i32),
        pltpu.VMEM((B, 16), f32),
        pltpu.VMEM_SHARED((N, 128), f32),
        pltpu.VMEM_SHARED((N, 16), f32),
        pltpu.SemaphoreType.DMA,
        pltpu.SemaphoreType.DMA,
        pltpu.SemaphoreType.DMA,
        pltpu.SemaphoreType.DMA,
    ],
)
def _sc_layer2(h2p, s1r, s2r, srcp, dstp, zf, zr, p0f, p0r, p1f, p1r,
               srcb0, dstb0, s1v0, s2v0, gbuf0,
               srcb1, dstb1, s1v1, s2v1, gbuf1,
               wv0, sbuf, wrbuf, acc, accr, semI0, semI1, semG0, semG1):
    cid = lax.axis_index("c")
    sid = lax.axis_index("s")
    _zero_rows(wrbuf, B)
    _node_copy(zf, acc, sid)
    _node_copy(zr, accr, sid)
    plsc.subcore_barrier()
    epc = E // NC
    ept = epc // NS
    NB = (ept + B - 1) // B
    ebase = cid * epc + sid * ept
    S0 = (srcb0, dstb0, s1v0, s2v0, gbuf0, semI0, semG0)
    S1 = (srcb1, dstb1, s1v1, s2v1, gbuf1, semI1, semG1)

    def base_of(b):
        return pl.multiple_of(ebase + b * B, 8)

    def idx_issue(b, s):
        base = base_of(b)
        pltpu.async_copy(srcp.at[pl.ds(base, B)], s[0], s[5])
        pltpu.async_copy(dstp.at[pl.ds(base, B)], s[1], s[5])

    def idx_wait(b, s):
        base = base_of(b)
        pltpu.make_async_copy(srcp.at[pl.ds(base, B)], s[0], s[5]).wait()
        pltpu.make_async_copy(dstp.at[pl.ds(base, B)], s[1], s[5]).wait()

    def g_issue(s):
        srcb, dstb, s1v, s2v, gbuf, _, semG = s
        pltpu.async_copy(h2p.at[dstb], gbuf, semG)
        pltpu.async_copy(s1r.at[srcb], s1v, semG)
        pltpu.async_copy(s2r.at[dstb], s2v, semG)

    def g_wait(s):
        srcb, dstb, s1v, s2v, gbuf, _, semG = s
        pltpu.make_async_copy(h2p.at[dstb], gbuf, semG).wait()
        pltpu.make_async_copy(s1r.at[srcb], s1v, semG).wait()
        pltpu.make_async_copy(s2r.at[dstb], s2v, semG).wait()

    def process(b, s):
        _, _, s1v, s2v, gbuf, _, _ = s
        thresh = jnp.minimum(B, ept - b * B)
        for g in range(B // 16):
            sl = pl.ds(g * 16, 16)
            rows = lax.iota(i32, 16) + g * 16
            w = _edge_weight(s1v[sl] + s2v[sl])
            w = jnp.where(rows < thresh, w, jnp.zeros((16,), f32))
            wv0[sl] = w
            plsc.store_scatter(wrbuf, [rows, _splat(0)], w)

        def srow(r, carry):
            a0 = plsc.load_gather(wv0, [_splat(r)])
            for j in range(8):
                sl = pl.ds(j * 16, 16)
                gbuf[r, sl] = gbuf[r, sl] * a0
            return carry

        lax.fori_loop(0, B, srow, None, unroll=4)
        pltpu.sync_copy(gbuf, acc.at[sbuf], add=True)
        pltpu.sync_copy(wrbuf, accr.at[sbuf], add=True)

    idx_issue(0, S0)
    idx_issue(1, S1)
    idx_wait(0, S0)
    g_issue(S0)

    def pair(i, carry):
        b0 = 2 * i
        g_wait(S0)
        _copy_idx(sbuf, S0[0])
        idx_issue(b0 + 2, S0)
        idx_wait(b0 + 1, S1)
        g_issue(S1)
        process(b0, S0)
        g_wait(S1)
        _copy_idx(sbuf, S1[0])
        idx_issue(b0 + 3, S1)
        idx_wait(b0 + 2, S0)
        g_issue(S0)
        process(b0 + 1, S1)
        return carry

    lax.fori_loop(0, NB // 2, pair, None)
    g_wait(S0)
    _copy_idx(sbuf, S0[0])
    process(NB - 1, S0)
    idx_wait(NB, S1)
    plsc.subcore_barrier()
    for k in range(NC):
        @pl.when(cid == k)
        def _(k=k):
            _node_copy(acc, (p0f, p1f)[k], sid)
            _node_copy(accr, (p0r, p1r)[k], sid)


def kernel(x, adj, W_att, a_att, W_out, a_out):
    src = adj[0]
    dst = adj[1]
    pad = jnp.zeros((EP - E,), i32)
    srcp = jnp.concatenate([src, pad])
    dstp = jnp.concatenate([dst, pad])

    wall = jnp.transpose(W_att, (1, 0, 2)).reshape(F_IN, NH * NHID)
    a1 = a_att[:, 0, :NHID]
    a2 = a_att[:, 0, NHID:]
    eye = jnp.eye(NH, dtype=f32)
    s1m = (a1[:, :, None] * eye[:, None, :]).reshape(NH * NHID, NH)
    s2m = (a2[:, :, None] * eye[:, None, :]).reshape(NH * NHID, NH)
    smat = jnp.concatenate([s1m, s2m], axis=1)

    v1 = W_out @ a_out[0, :NLABEL]
    v2 = W_out @ a_out[0, NLABEL:]
    wbig = jnp.concatenate(
        [W_out, jnp.zeros((NH * NHID, 7), f32), v2[:, None], v1[:, None],
         jnp.zeros((NH * NHID, 14), f32)], axis=1)

    zf = jnp.zeros((N, 128), f32)
    zr = jnp.zeros((N, 16), f32)

    h0, h1, h2, h3, s = _tc1(x, wall, smat)
    sts = [s[:, i] for i in range(16)]
    of0, or0, of1, or1 = _sc_layer1(
        h0, h1, sts[0], sts[1], sts[8], sts[9],
        sts[2], sts[3], sts[10], sts[11], srcp, dstp, zf, zr)
    of2, or2, of3, or3 = _sc_layer1(
        h2, h3, sts[4], sts[5], sts[12], sts[13],
        sts[6], sts[7], sts[14], sts[15], srcp, dstp, zf, zr)
    h2p, sv = _tc2(of0, of1, of2, of3, or0, or1, or2, or3, wbig)
    s2r = sv[:, 0]
    s1r = sv[:, 1]
    p0f, p0r, p1f, p1r = _sc_layer2(h2p, s1r, s2r, srcp, dstp, zf, zr)
    full = _tc3(p0f, p1f, p0r, p1r)
    return full[:, :NLABEL]
